# baseline (device time: 108464 ns/iter reference)
import jax
import jax.numpy as jnp
from jax import lax
from jax.experimental import pallas as pl
from jax.experimental.pallas import tpu as pltpu

N_DEV = 8
M_PER = 512
NSUB = 2
SUB = M_PER // NSUB

FA, BA, Z, FB, FD, BC, BD = range(7)


def kernel(x, w_mat, scale_x, scale_w):
    m_per, k = x.shape
    _, n_per = w_mat.shape
    scale = (scale_x[0] * scale_w[0]).reshape(1, 1)

    def body(x_ref, w_ref, scale_ref, out_ref, allx_ref,
             vstage_ref, send_sems, recv_sems, copy_sems):
        my = lax.axis_index("i")
        j4 = lax.rem(my, 4)
        base = my - j4
        right = base + lax.rem(j4 + 1, 4)
        left = base + lax.rem(j4 + 3, 4)
        anti = lax.rem(my + 4, N_DEV)
        ll = base + lax.rem(j4 + 2, 4)
        l_anti = lax.rem(left + 4, N_DEV)
        r_anti = lax.rem(right + 4, N_DEV)
        rr_anti = lax.rem(ll + 4, N_DEV)

        barrier = pltpu.get_barrier_semaphore()
        for nbr in (left, right, anti):
            pl.semaphore_signal(barrier, inc=1, device_id=(nbr,),
                                device_id_type=pl.DeviceIdType.MESH)
        pl.semaphore_wait(barrier, 3)

        def gemm(chunk, row0):
            acc = lax.dot_general(
                chunk.astype(jnp.bfloat16), w_ref[...].astype(jnp.bfloat16),
                dimension_numbers=(((1,), (0,)), ((), ())),
                preferred_element_type=jnp.float32,
            )
            out_ref[pl.ds(row0, M_PER), :] = acc * scale_ref[0, 0]

        def rdma(origin, flow, s, dst, src_ref=None):
            sl = pl.ds(s * SUB, SUB)
            return pltpu.make_async_remote_copy(
                src_ref=allx_ref.at[origin, sl] if src_ref is None
                else src_ref.at[sl],
                dst_ref=allx_ref.at[origin, sl],
                send_sem=send_sems.at[flow, s], recv_sem=recv_sems.at[flow, s],
                device_id=(dst,), device_id_type=pl.DeviceIdType.MESH,
            )

        for s in range(NSUB):
            rdma(my, FA, s, right, src_ref=x_ref).start()
            rdma(my, BA, s, left, src_ref=x_ref).start()
            rdma(my, Z, s, anti, src_ref=x_ref).start()

        gemm(x_ref[...], my * M_PER)

        pending = []

        def stage(origin, slot):
            pltpu.make_async_copy(
                allx_ref.at[origin], vstage_ref.at[slot], copy_sems.at[slot],
            ).start()
            pending.append((origin, slot))

        def pop_gemm():
            origin, slot = pending.pop(0)
            pltpu.make_async_copy(
                allx_ref.at[origin], vstage_ref.at[slot], copy_sems.at[slot],
            ).wait()
            gemm(vstage_ref[slot], origin * M_PER)

        for s in range(NSUB):
            rdma(left, FA, s, right).wait_recv()
            rdma(left, FB, s, right).start()
        stage(left, 0)

        for s in range(NSUB):
            rdma(anti, Z, s, anti).wait_recv()
            rdma(anti, FD, s, right).start()
            rdma(anti, BC, s, left).start()
        stage(anti, 1)
        pop_gemm()

        for s in range(NSUB):
            rdma(right, BA, s, left).wait_recv()
        stage(right, 0)
        pop_gemm()

        for s in range(NSUB):
            rdma(r_anti, BC, s, left).wait_recv()
            rdma(r_anti, BD, s, left).start()
        stage(r_anti, 1)
        pop_gemm()

        for s in range(NSUB):
            rdma(ll, FB, s, right).wait_recv()
        stage(ll, 0)
        pop_gemm()

        for s in range(NSUB):
            rdma(l_anti, FD, s, right).wait_recv()
        stage(l_anti, 1)
        pop_gemm()

        for s in range(NSUB):
            rdma(rr_anti, BD, s, left).wait_recv()
        stage(rr_anti, 0)
        pop_gemm()
        pop_gemm()

        for s in range(NSUB):
            rdma(my, FA, s, right, src_ref=x_ref).wait_send()
            rdma(my, BA, s, left, src_ref=x_ref).wait_send()
            rdma(my, Z, s, anti, src_ref=x_ref).wait_send()
            rdma(left, FB, s, right).wait_send()
            rdma(anti, FD, s, right).wait_send()
            rdma(anti, BC, s, left).wait_send()
            rdma(r_anti, BD, s, left).wait_send()

    out, _ = pl.pallas_call(
        body,
        out_shape=(
            jax.ShapeDtypeStruct((N_DEV * m_per, n_per), jnp.float32),
            jax.ShapeDtypeStruct((N_DEV, m_per, k), jnp.int8),
        ),
        in_specs=[
            pl.BlockSpec(memory_space=pltpu.VMEM),
            pl.BlockSpec(memory_space=pltpu.VMEM),
            pl.BlockSpec(memory_space=pltpu.SMEM),
        ],
        out_specs=(
            pl.BlockSpec(memory_space=pltpu.VMEM),
            pl.BlockSpec(memory_space=pl.ANY),
        ),
        scratch_shapes=[
            pltpu.VMEM((2, M_PER, k), jnp.int8),
            pltpu.SemaphoreType.DMA((7, NSUB)),
            pltpu.SemaphoreType.DMA((7, NSUB)),
            pltpu.SemaphoreType.DMA((2,)),
        ],
        compiler_params=pltpu.CompilerParams(
            collective_id=0, vmem_limit_bytes=100 * 1024 * 1024,
        ),
    )(x, w_mat, scale)
    return out


# device time: 105601 ns/iter; 1.0271x vs baseline; 1.0271x over previous
import jax
import jax.numpy as jnp
from jax import lax
from jax.experimental import pallas as pl
from jax.experimental.pallas import tpu as pltpu

N_DEV = 8
M_PER = 512
NSUB = 4
SUB = M_PER // NSUB

FA, BA, Z, FB, FD, BC, BD = range(7)


def kernel(x, w_mat, scale_x, scale_w):
    m_per, k = x.shape
    _, n_per = w_mat.shape
    scale = (scale_x[0] * scale_w[0]).reshape(1, 1)

    def body(x_ref, w_ref, scale_ref, out_ref, allx_ref, send_sems, recv_sems):
        my = lax.axis_index("i")
        j4 = lax.rem(my, 4)
        base = my - j4
        right = base + lax.rem(j4 + 1, 4)
        left = base + lax.rem(j4 + 3, 4)
        anti = lax.rem(my + 4, N_DEV)
        ll = base + lax.rem(j4 + 2, 4)
        l_anti = lax.rem(left + 4, N_DEV)
        r_anti = lax.rem(right + 4, N_DEV)
        rr_anti = lax.rem(ll + 4, N_DEV)

        barrier = pltpu.get_barrier_semaphore()
        for nbr in (left, right, anti):
            pl.semaphore_signal(barrier, inc=1, device_id=(nbr,),
                                device_id_type=pl.DeviceIdType.MESH)
        pl.semaphore_wait(barrier, 3)

        def gemm(origin):
            acc = lax.dot_general(
                allx_ref[origin].reshape(M_PER, k).astype(jnp.bfloat16),
                w_ref[...].astype(jnp.bfloat16),
                dimension_numbers=(((1,), (0,)), ((), ())),
                preferred_element_type=jnp.float32,
            )
            out_ref[pl.ds(origin * M_PER, M_PER), :] = acc * scale_ref[0, 0]

        def rdma(origin, flow, s, dst):
            return pltpu.make_async_remote_copy(
                src_ref=allx_ref.at[origin, s], dst_ref=allx_ref.at[origin, s],
                send_sem=send_sems.at[flow, s], recv_sem=recv_sems.at[flow, s],
                device_id=(dst,), device_id_type=pl.DeviceIdType.MESH,
            )

        allx_ref[my] = x_ref[...].reshape(NSUB, SUB, k)

        for s in range(NSUB):
            rdma(my, Z, s, anti).start()
            rdma(my, FA, s, right).start()
            rdma(my, BA, s, left).start()

        acc = lax.dot_general(
            x_ref[...].astype(jnp.bfloat16), w_ref[...].astype(jnp.bfloat16),
            dimension_numbers=(((1,), (0,)), ((), ())),
            preferred_element_type=jnp.float32,
        )
        out_ref[pl.ds(my * M_PER, M_PER), :] = acc * scale_ref[0, 0]

        for s in range(NSUB):
            rdma(left, FA, s, right).wait_recv()
            rdma(left, FB, s, right).start()
        gemm(left)

        for s in range(NSUB):
            rdma(anti, Z, s, anti).wait_recv()
            rdma(anti, BC, s, left).start()
            rdma(anti, FD, s, right).start()
        gemm(anti)

        for s in range(NSUB):
            rdma(right, BA, s, left).wait_recv()
        gemm(right)

        for s in range(NSUB):
            rdma(r_anti, BC, s, left).wait_recv()
            rdma(r_anti, BD, s, left).start()
        gemm(r_anti)

        for s in range(NSUB):
            rdma(ll, FB, s, right).wait_recv()
        gemm(ll)

        for s in range(NSUB):
            rdma(l_anti, FD, s, right).wait_recv()
        gemm(l_anti)

        for s in range(NSUB):
            rdma(rr_anti, BD, s, left).wait_recv()
        gemm(rr_anti)

        for s in range(NSUB):
            rdma(my, FA, s, right).wait_send()
            rdma(my, BA, s, left).wait_send()
            rdma(my, Z, s, anti).wait_send()
            rdma(left, FB, s, right).wait_send()
            rdma(anti, FD, s, right).wait_send()
            rdma(anti, BC, s, left).wait_send()
            rdma(r_anti, BD, s, left).wait_send()

    return pl.pallas_call(
        body,
        out_shape=jax.ShapeDtypeStruct((N_DEV * m_per, n_per), jnp.float32),
        in_specs=[
            pl.BlockSpec(memory_space=pltpu.VMEM),
            pl.BlockSpec(memory_space=pltpu.VMEM),
            pl.BlockSpec(memory_space=pltpu.SMEM),
        ],
        out_specs=pl.BlockSpec(memory_space=pltpu.VMEM),
        scratch_shapes=[
            pltpu.VMEM((N_DEV, NSUB, SUB, k), jnp.int8),
            pltpu.SemaphoreType.DMA((7, NSUB)),
            pltpu.SemaphoreType.DMA((7, NSUB)),
        ],
        compiler_params=pltpu.CompilerParams(
            collective_id=0, vmem_limit_bytes=100 * 1024 * 1024,
        ),
    )(x, w_mat, scale)
